# Initial kernel scaffold; baseline (speedup 1.0000x reference)
import jax
import jax.numpy as jnp
from jax.experimental import pallas as pl


def kernel(all_node_reprs, ts, neigh_ts_l2, neigh_ts_l1, node_emb, edge_emb, time_w, time_b, Wq1, Wk1, Wv1, W1_1, b1_1, W2_1, b2_1, Wq2, Wk2, Wv2, W1_2, b1_2, W2_2, b2_2, center_nids, neigh_nids_l2, neigh_eids_l2, neigh_nids_l1, neigh_eids_l1):
    raise NotImplementedError("write your pallas kernel here")


# speedup vs baseline: 4.0876x; 4.0876x over previous
import functools

import jax
import jax.numpy as jnp
import numpy as np
from jax import lax
from jax.experimental import pallas as pl
from jax.experimental.pallas import tpu as pltpu
from jax.experimental.pallas import tpu_sc as plsc

N_NODES = 50000
N_EDGES = 800000
B = 512
NB = 20
D = 128
DE = 16
DT = 128
DH = 128

_NC = 2
_NS = 16
_NW = _NC * _NS
_CH = 128

_N_GATHER_NODES = B * NB * NB + B * NB + B
_N_GATHER_EDGES = B * NB * NB + B * NB
_CHUNKS = -(-max(_N_GATHER_NODES, _N_GATHER_EDGES) // (_NW * _CH))
_PAD_LEN = _CHUNKS * _NW * _CH
_PER_W = _CHUNKS * _CH


def _add_body(a_ref, b_ref, o_ref):
    o_ref[...] = a_ref[...] + b_ref[...]


def _combined_table(a, b):
    blk = 2000
    return pl.pallas_call(
        _add_body,
        grid=(N_NODES // blk,),
        in_specs=[pl.BlockSpec((blk, D), lambda i: (i, 0))] * 2,
        out_specs=pl.BlockSpec((blk, D), lambda i: (i, 0)),
        out_shape=jax.ShapeDtypeStruct((N_NODES, D), jnp.float32),
    )(a, b)


def _sc_gather_body(node_tab, edge_tab, nidx, eidx, node_out, edge_out,
                    nidx_v, nrow_v, eidx_v, erow_v, sem_n, sem_e):
    wid = lax.axis_index("s") * _NC + lax.axis_index("c")
    base0 = wid * _PER_W

    def chunk(it, carry):
        base = base0 + it * _CH
        pltpu.sync_copy(nidx.at[pl.ds(base, _CH)], nidx_v)
        cp_n = pltpu.async_copy(node_tab.at[nidx_v], nrow_v, sem_n)
        pltpu.sync_copy(eidx.at[pl.ds(base, _CH)], eidx_v)
        cp_e = pltpu.async_copy(edge_tab.at[eidx_v], erow_v, sem_e)
        cp_n.wait()
        pltpu.sync_copy(nrow_v, node_out.at[pl.ds(base, _CH)])
        cp_e.wait()
        pltpu.sync_copy(erow_v, edge_out.at[pl.ds(base, _CH)])
        return carry

    lax.fori_loop(0, _CHUNKS, chunk, 0)


def _sc_gather(node_tab, edge_tab, nidx, eidx):
    mesh = plsc.VectorSubcoreMesh(core_axis_name="c", subcore_axis_name="s")
    fn = functools.partial(
        pl.kernel,
        mesh=mesh,
        compiler_params=pltpu.CompilerParams(use_tc_tiling_on_sc=False),
        out_type=[
            jax.ShapeDtypeStruct((_PAD_LEN, D), jnp.float32),
            jax.ShapeDtypeStruct((_PAD_LEN, DE), jnp.float32),
        ],
        scratch_types=[
            pltpu.VMEM((_CH,), jnp.int32),
            pltpu.VMEM((_CH, D), jnp.float32),
            pltpu.VMEM((_CH,), jnp.int32),
            pltpu.VMEM((_CH, DE), jnp.float32),
            pltpu.SemaphoreType.DMA,
            pltpu.SemaphoreType.DMA,
        ],
    )(_sc_gather_body)
    return fn(node_tab, edge_tab, nidx, eidx)


def _attn_body(msz, leaf_ref, edge_ref, c_ref, nts_ref, nid_ref, tsc_ref,
               wq_ref, wk_ref, wv_ref, w1_ref, b1_ref, w2_ref, b2_ref,
               tw_ref, tb_ref, o_ref):
    f32 = jnp.float32
    dot = functools.partial(jnp.dot, preferred_element_type=f32)
    tw = tw_ref[...]
    tb = tb_ref[...]
    cb = c_ref[...]
    q = dot(cb, wq_ref[:D, :]) + dot(jnp.cos(tb), wq_ref[D:, :])
    tscol = tsc_ref[...]
    m = jnp.full((msz, 1), -1e30, f32)
    s = jnp.zeros((msz, 1), f32)
    acc = jnp.zeros((msz, DH), f32)
    inv_sqrt = f32(1.0 / np.sqrt(DH))
    for j in range(NB):
        lf = leaf_ref[j]
        eg = edge_ref[j]
        dcol = tscol - nts_ref[:, j:j + 1]
        dtx = jnp.cos(dcol * tw + tb)
        k = (dot(lf, wk_ref[:D, :]) + dot(eg, wk_ref[D:D + DE, :])
             + dot(dtx, wk_ref[D + DE:, :]))
        v = (dot(lf, wv_ref[:D, :]) + dot(eg, wv_ref[D:D + DE, :])
             + dot(dtx, wv_ref[D + DE:, :]))
        sj = jnp.sum(q * k, axis=1, keepdims=True) * inv_sqrt
        sj = jnp.where(nid_ref[:, j:j + 1] == 0, f32(-1e10), sj)
        mn = jnp.maximum(m, sj)
        cscale = jnp.exp(m - mn)
        p = jnp.exp(sj - mn)
        s = s * cscale + p
        acc = acc * cscale + p * v
        m = mn
    agg = acc / s
    hmid = jnp.maximum(dot(agg, w1_ref[:DH, :]) + dot(cb, w1_ref[DH:, :])
                       + b1_ref[...], 0.0)
    o_ref[...] = dot(hmid, w2_ref[...]) + b2_ref[...]


def _attn_level(leafT, edgeT, cx, nts, nid, tscol, wq, wk, wv, w1, b1, w2, b2,
                tw, tb, msz):
    n = cx.shape[0]
    grid = (n // msz,)
    wspec = lambda shape: pl.BlockSpec(shape, lambda i: (0, 0))
    return pl.pallas_call(
        functools.partial(_attn_body, msz),
        grid=grid,
        in_specs=[
            pl.BlockSpec((NB, msz, D), lambda i: (0, i, 0)),
            pl.BlockSpec((NB, msz, DE), lambda i: (0, i, 0)),
            pl.BlockSpec((msz, D), lambda i: (i, 0)),
            pl.BlockSpec((msz, NB), lambda i: (i, 0)),
            pl.BlockSpec((msz, NB), lambda i: (i, 0)),
            pl.BlockSpec((msz, 1), lambda i: (i, 0)),
            wspec((D + DT, DH)),
            wspec((D + DE + DT, DH)),
            wspec((D + DE + DT, DH)),
            wspec((DH + D, DH)),
            wspec((1, DH)),
            wspec((DH, D)),
            wspec((1, D)),
            wspec((1, DT)),
            wspec((1, DT)),
        ],
        out_specs=pl.BlockSpec((msz, D), lambda i: (i, 0)),
        out_shape=jax.ShapeDtypeStruct((n, D), jnp.float32),
    )(leafT, edgeT, cx, nts, nid, tscol, wq, wk, wv, w1, b1, w2, b2, tw, tb)


def kernel(all_node_reprs, ts, neigh_ts_l2, neigh_ts_l1, node_emb, edge_emb,
           time_w, time_b, Wq1, Wk1, Wv1, W1_1, b1_1, W2_1, b2_1,
           Wq2, Wk2, Wv2, W1_2, b1_2, W2_2, b2_2,
           center_nids, neigh_nids_l2, neigh_eids_l2,
           neigh_nids_l1, neigh_eids_l1):
    f32 = jnp.float32
    i32 = jnp.int32

    nids1_3d = neigh_nids_l1.reshape(B, NB, NB)
    eids1_3d = neigh_eids_l1.reshape(B, NB, NB)
    nts1_3d = neigh_ts_l1.reshape(B, NB, NB)
    leaf_list = nids1_3d.transpose(2, 1, 0).reshape(-1)
    e1_list = eids1_3d.transpose(2, 1, 0).reshape(-1)
    c1_list = neigh_nids_l2.T.reshape(-1)
    e2_list = neigh_eids_l2.T.reshape(-1)
    nts1 = nts1_3d.transpose(1, 0, 2).reshape(B * NB, NB)
    nid1 = nids1_3d.transpose(1, 0, 2).reshape(B * NB, NB)
    ts1 = jnp.tile(ts, NB).reshape(B * NB, 1)

    node_list = jnp.concatenate([leaf_list, c1_list, center_nids.astype(i32)])
    node_list = jnp.concatenate(
        [node_list, jnp.zeros((_PAD_LEN - _N_GATHER_NODES,), i32)])
    edge_list = jnp.concatenate([e1_list, e2_list])
    edge_list = jnp.concatenate(
        [edge_list, jnp.zeros((_PAD_LEN - _N_GATHER_EDGES,), i32)])

    tw2 = time_w.reshape(1, DT)
    tb2 = time_b.reshape(1, DT)
    b1_1r = b1_1.reshape(1, DH)
    b2_1r = b2_1.reshape(1, D)
    b1_2r = b1_2.reshape(1, DH)
    b2_2r = b2_2.reshape(1, D)

    comb = _combined_table(all_node_reprs, node_emb)

    node_rows, edge_rows = _sc_gather(comb, edge_emb, node_list, edge_list)

    leafT = node_rows[:B * NB * NB].reshape(NB, B * NB, D)
    c1 = node_rows[B * NB * NB:B * NB * NB + B * NB]
    c2 = node_rows[B * NB * NB + B * NB:_N_GATHER_NODES]
    e1T = edge_rows[:B * NB * NB].reshape(NB, B * NB, DE)
    e2T = edge_rows[B * NB * NB:_N_GATHER_EDGES].reshape(NB, B, DE)

    h1 = _attn_level(leafT, e1T, c1, nts1, nid1, ts1,
                     Wq1, Wk1, Wv1, W1_1, b1_1r, W2_1, b2_1r, tw2, tb2,
                     msz=256)

    neigh2T = h1.reshape(NB, B, D)
    out = _attn_level(neigh2T, e2T, c2, neigh_ts_l2, neigh_nids_l2,
                      ts.reshape(B, 1),
                      Wq2, Wk2, Wv2, W1_2, b1_2r, W2_2, b2_2r, tw2, tb2,
                      msz=B)
    return out


# speedup vs baseline: 4.1041x; 1.0040x over previous
import functools

import jax
import jax.numpy as jnp
import numpy as np
from jax import lax
from jax.experimental import pallas as pl
from jax.experimental.pallas import tpu as pltpu
from jax.experimental.pallas import tpu_sc as plsc

N_NODES = 50000
N_EDGES = 800000
B = 512
NB = 20
D = 128
DE = 16
DT = 128
DH = 128

_NC = 2
_NS = 16
_NW = _NC * _NS
_CH = 128

_N_LEAF = B * NB * NB
_LEAF_PER_W = _N_LEAF // _NW
_LEAF_CHUNKS = _LEAF_PER_W // _CH
_N_C1 = B * NB
_C1_PAD = _NW * 3 * _CH
_C1_PER_W = _C1_PAD // _NW
_C2_PER_W = B // _NW


def _add_body(a_ref, b_ref, o_ref):
    o_ref[...] = (a_ref[...] + b_ref[...]).astype(jnp.bfloat16)


def _combined_table(a, b):
    blk = 2000
    return pl.pallas_call(
        _add_body,
        grid=(N_NODES // blk,),
        in_specs=[pl.BlockSpec((blk, D), lambda i: (i, 0))] * 2,
        out_specs=pl.BlockSpec((blk, D), lambda i: (i, 0)),
        out_shape=jax.ShapeDtypeStruct((N_NODES, D), jnp.bfloat16),
    )(a, b)


def _sc_gather_body(ntab, etab, leaf_idx, e1_idx, c1_idx, e2_idx, c2_idx,
                    leaf_out, e1_out, c1_out, e2_out, c2_out,
                    nidx_v, nrow_v, eidx_v, erow_v, cidx_v, crow_v,
                    sem_n, sem_e, sem_c):
    wid = lax.axis_index("s") * _NC + lax.axis_index("c")
    nb0 = wid * _LEAF_PER_W

    def issue(b, cur):
        base = nb0 + cur * _CH
        pltpu.sync_copy(leaf_idx.at[pl.ds(base, _CH)], nidx_v.at[b])
        pltpu.async_copy(ntab.at[nidx_v.at[b]], nrow_v.at[b], sem_n)
        pltpu.sync_copy(e1_idx.at[pl.ds(base, _CH)], eidx_v.at[b])
        pltpu.async_copy(etab.at[eidx_v.at[b]], erow_v.at[b], sem_e)

    issue(0, 0)
    issue(1, 1)

    def outer(it, carry):
        for bsel in (0, 1):
            cur = 2 * it + bsel
            base = nb0 + cur * _CH
            pltpu.make_async_copy(ntab.at[nidx_v.at[bsel]], nrow_v.at[bsel],
                                  sem_n).wait()
            pltpu.sync_copy(nrow_v.at[bsel], leaf_out.at[pl.ds(base, _CH)])
            pltpu.make_async_copy(etab.at[eidx_v.at[bsel]], erow_v.at[bsel],
                                  sem_e).wait()
            pltpu.sync_copy(erow_v.at[bsel], e1_out.at[pl.ds(base, _CH)])
            nxt = cur + 2

            @pl.when(nxt < _LEAF_CHUNKS)
            def _():
                issue(bsel, nxt)

        return carry

    lax.fori_loop(0, _LEAF_CHUNKS // 2, outer, 0)

    c1b0 = wid * _C1_PER_W
    for t in range(3):
        base = c1b0 + t * _CH
        pltpu.sync_copy(c1_idx.at[pl.ds(base, _CH)], nidx_v.at[0])
        pltpu.async_copy(ntab.at[nidx_v.at[0]], nrow_v.at[0], sem_n).wait()
        pltpu.sync_copy(nrow_v.at[0], c1_out.at[pl.ds(base, _CH)])
        pltpu.sync_copy(e2_idx.at[pl.ds(base, _CH)], eidx_v.at[0])
        pltpu.async_copy(etab.at[eidx_v.at[0]], erow_v.at[0], sem_e).wait()
        pltpu.sync_copy(erow_v.at[0], e2_out.at[pl.ds(base, _CH)])

    c2b = wid * _C2_PER_W
    pltpu.sync_copy(c2_idx.at[pl.ds(c2b, _C2_PER_W)], cidx_v)
    pltpu.async_copy(ntab.at[cidx_v], crow_v, sem_c).wait()
    pltpu.sync_copy(crow_v, c2_out.at[pl.ds(c2b, _C2_PER_W)])


def _sc_gather(ntab, etab, leaf_idx, e1_idx, c1_idx, e2_idx, c2_idx):
    bf16 = jnp.bfloat16
    f32 = jnp.float32
    mesh = plsc.VectorSubcoreMesh(core_axis_name="c", subcore_axis_name="s")
    fn = functools.partial(
        pl.kernel,
        mesh=mesh,
        compiler_params=pltpu.CompilerParams(use_tc_tiling_on_sc=False),
        out_type=[
            jax.ShapeDtypeStruct((_N_LEAF, D), bf16),
            jax.ShapeDtypeStruct((_N_LEAF, DE), f32),
            jax.ShapeDtypeStruct((_C1_PAD, D), bf16),
            jax.ShapeDtypeStruct((_C1_PAD, DE), f32),
            jax.ShapeDtypeStruct((B, D), bf16),
        ],
        scratch_types=[
            pltpu.VMEM((2, _CH), jnp.int32),
            pltpu.VMEM((2, _CH, D), bf16),
            pltpu.VMEM((2, _CH), jnp.int32),
            pltpu.VMEM((2, _CH, DE), f32),
            pltpu.VMEM((_C2_PER_W,), jnp.int32),
            pltpu.VMEM((_C2_PER_W, D), bf16),
            pltpu.SemaphoreType.DMA,
            pltpu.SemaphoreType.DMA,
            pltpu.SemaphoreType.DMA,
        ],
    )(_sc_gather_body)
    return fn(ntab, etab, leaf_idx, e1_idx, c1_idx, e2_idx, c2_idx)


def _attn_body(msz, cdt, leaf_ref, edge_ref, c_ref, nts_ref, nid_ref, tsc_ref,
               wq_ref, wk_ref, wv_ref, w1_ref, b1_ref, w2_ref, b2_ref,
               tw_ref, tb_ref, o_ref):
    f32 = jnp.float32
    dot = functools.partial(jnp.dot, preferred_element_type=f32)
    tw = tw_ref[...]
    tb = tb_ref[...]
    cb = c_ref[...].astype(cdt)
    q = dot(cb, wq_ref[:D, :]) + dot(jnp.cos(tb).astype(cdt), wq_ref[D:, :])
    tscol = tsc_ref[...]
    m = jnp.full((msz, 1), -1e30, f32)
    s = jnp.zeros((msz, 1), f32)
    acc = jnp.zeros((msz, DH), f32)
    inv_sqrt = f32(1.0 / np.sqrt(DH))
    for j in range(NB):
        lf = leaf_ref[j].astype(cdt)
        eg = edge_ref[j].astype(cdt)
        dcol = tscol - nts_ref[:, j:j + 1]
        dtx = jnp.cos(dcol * tw + tb).astype(cdt)
        k = (dot(lf, wk_ref[:D, :]) + dot(eg, wk_ref[D:D + DE, :])
             + dot(dtx, wk_ref[D + DE:, :]))
        v = (dot(lf, wv_ref[:D, :]) + dot(eg, wv_ref[D:D + DE, :])
             + dot(dtx, wv_ref[D + DE:, :]))
        sj = jnp.sum(q * k, axis=1, keepdims=True) * inv_sqrt
        sj = jnp.where(nid_ref[:, j:j + 1] == 0, f32(-1e10), sj)
        mn = jnp.maximum(m, sj)
        cscale = jnp.exp(m - mn)
        p = jnp.exp(sj - mn)
        s = s * cscale + p
        acc = acc * cscale + p * v
        m = mn
    agg = (acc / s).astype(cdt)
    hmid = jnp.maximum(dot(agg, w1_ref[:DH, :]) + dot(cb, w1_ref[DH:, :])
                       + b1_ref[...], 0.0).astype(cdt)
    o_ref[...] = (dot(hmid, w2_ref[...]) + b2_ref[...]).astype(o_ref.dtype)


def _attn_level(leafT, edgeT, cx, nts, nid, tscol, wq, wk, wv, w1, b1, w2, b2,
                tw, tb, msz, n, cdt, odt):
    grid = (n // msz,)
    wspec = lambda shape: pl.BlockSpec(shape, lambda i: (0, 0))
    return pl.pallas_call(
        functools.partial(_attn_body, msz, cdt),
        grid=grid,
        in_specs=[
            pl.BlockSpec((NB, msz, D), lambda i: (0, i, 0)),
            pl.BlockSpec((NB, msz, DE), lambda i: (0, i, 0)),
            pl.BlockSpec((msz, D), lambda i: (i, 0)),
            pl.BlockSpec((msz, NB), lambda i: (i, 0)),
            pl.BlockSpec((msz, NB), lambda i: (i, 0)),
            pl.BlockSpec((msz, 1), lambda i: (i, 0)),
            wspec((D + DT, DH)),
            wspec((D + DE + DT, DH)),
            wspec((D + DE + DT, DH)),
            wspec((DH + D, DH)),
            wspec((1, DH)),
            wspec((DH, D)),
            wspec((1, D)),
            wspec((1, DT)),
            wspec((1, DT)),
        ],
        out_specs=pl.BlockSpec((msz, D), lambda i: (i, 0)),
        out_shape=jax.ShapeDtypeStruct((n, D), odt),
    )(leafT, edgeT, cx, nts, nid, tscol, wq, wk, wv, w1, b1, w2, b2, tw, tb)


def kernel(all_node_reprs, ts, neigh_ts_l2, neigh_ts_l1, node_emb, edge_emb,
           time_w, time_b, Wq1, Wk1, Wv1, W1_1, b1_1, W2_1, b2_1,
           Wq2, Wk2, Wv2, W1_2, b1_2, W2_2, b2_2,
           center_nids, neigh_nids_l2, neigh_eids_l2,
           neigh_nids_l1, neigh_eids_l1):
    f32 = jnp.float32
    i32 = jnp.int32
    bf16 = jnp.bfloat16

    nids1_3d = neigh_nids_l1.reshape(B, NB, NB)
    eids1_3d = neigh_eids_l1.reshape(B, NB, NB)
    nts1_3d = neigh_ts_l1.reshape(B, NB, NB)
    leaf_list = nids1_3d.transpose(2, 1, 0).reshape(-1)
    e1_list = eids1_3d.transpose(2, 1, 0).reshape(-1)
    c1_pad = jnp.zeros((_C1_PAD - _N_C1,), i32)
    c1_list = jnp.concatenate([neigh_nids_l2.T.reshape(-1), c1_pad])
    e2_list = jnp.concatenate([neigh_eids_l2.T.reshape(-1), c1_pad])
    nts1 = nts1_3d.transpose(1, 0, 2).reshape(B * NB, NB)
    nid1 = nids1_3d.transpose(1, 0, 2).reshape(B * NB, NB)
    ts1 = jnp.tile(ts, NB).reshape(B * NB, 1)

    tw2 = time_w.reshape(1, DT)
    tb2 = time_b.reshape(1, DT)
    b1_1r = b1_1.reshape(1, DH)
    b2_1r = b2_1.reshape(1, D)
    b1_2r = b1_2.reshape(1, DH)
    b2_2r = b2_2.reshape(1, D)

    comb = _combined_table(all_node_reprs, node_emb)

    leaf_rows, e1_rows, c1_rows, e2_rows, c2 = _sc_gather(
        comb, edge_emb, leaf_list, e1_list, c1_list, e2_list,
        center_nids.astype(i32))

    leafT = leaf_rows.reshape(NB, B * NB, D)
    e1T = e1_rows.reshape(NB, B * NB, DE)
    e2T = e2_rows[:_N_C1].reshape(NB, B, DE)

    h1 = _attn_level(leafT, e1T, c1_rows, nts1, nid1, ts1,
                     Wq1.astype(bf16), Wk1.astype(bf16), Wv1.astype(bf16),
                     W1_1.astype(bf16), b1_1r, W2_1.astype(bf16), b2_1r,
                     tw2, tb2, msz=256, n=B * NB, cdt=bf16, odt=bf16)

    neigh2T = h1.reshape(NB, B, D)
    out = _attn_level(neigh2T, e2T, c2, neigh_ts_l2, neigh_nids_l2,
                      ts.reshape(B, 1),
                      Wq2, Wk2, Wv2, W1_2, b1_2r, W2_2, b2_2r, tw2, tb2,
                      msz=B, n=B, cdt=f32, odt=f32)
    return out


# speedup vs baseline: 6.7031x; 1.6333x over previous
import functools

import jax
import jax.numpy as jnp
import numpy as np
from jax import lax
from jax.experimental import pallas as pl
from jax.experimental.pallas import tpu as pltpu
from jax.experimental.pallas import tpu_sc as plsc

N_NODES = 50000
N_EDGES = 800000
B = 512
NB = 20
D = 128
DE = 16
DT = 128
DH = 128

_NC = 2
_NS = 16
_NW = _NC * _NS
_CH = 128

_N_LEAF = B * NB * NB
_LEAF_PER_W = _N_LEAF // _NW
_LEAF_CHUNKS = _LEAF_PER_W // _CH
_N_C1 = B * NB
_C1_PAD = _NW * 3 * _CH
_C1_PER_W = _C1_PAD // _NW
_C2_PER_W = B // _NW


def _add_body(a_ref, b_ref, o_ref):
    o_ref[...] = (a_ref[...] + b_ref[...]).astype(jnp.bfloat16)


def _combined_table(a, b):
    blk = 2000
    return pl.pallas_call(
        _add_body,
        grid=(N_NODES // blk,),
        in_specs=[pl.BlockSpec((blk, D), lambda i: (i, 0))] * 2,
        out_specs=pl.BlockSpec((blk, D), lambda i: (i, 0)),
        out_shape=jax.ShapeDtypeStruct((N_NODES, D), jnp.bfloat16),
    )(a, b)


def _sc_gather_body(ntab, etab, leaf_idx, e1_idx, c1_idx, e2_idx, c2_idx,
                    leaf_out, e1_out, c1_out, e2_out, c2_out,
                    nidx_v, nrow_v, eidx_v, erow_v, cidx_v, crow_v,
                    sem_n, sem_e, sem_c):
    wid = lax.axis_index("s") * _NC + lax.axis_index("c")
    nb0 = wid * _LEAF_PER_W

    def issue(b, cur):
        base = nb0 + cur * _CH
        pltpu.sync_copy(leaf_idx.at[pl.ds(base, _CH)], nidx_v.at[b])
        pltpu.async_copy(ntab.at[nidx_v.at[b]], nrow_v.at[b], sem_n)
        pltpu.sync_copy(e1_idx.at[pl.ds(base, _CH)], eidx_v.at[b])
        pltpu.async_copy(etab.at[eidx_v.at[b]], erow_v.at[b], sem_e)

    issue(0, 0)
    issue(1, 1)

    def outer(it, carry):
        for bsel in (0, 1):
            cur = 2 * it + bsel
            base = nb0 + cur * _CH
            pltpu.make_async_copy(ntab.at[nidx_v.at[bsel]], nrow_v.at[bsel],
                                  sem_n).wait()
            pltpu.sync_copy(nrow_v.at[bsel], leaf_out.at[pl.ds(base, _CH)])
            pltpu.make_async_copy(etab.at[eidx_v.at[bsel]], erow_v.at[bsel],
                                  sem_e).wait()
            pltpu.sync_copy(erow_v.at[bsel], e1_out.at[pl.ds(base, _CH)])
            nxt = cur + 2

            @pl.when(nxt < _LEAF_CHUNKS)
            def _():
                issue(bsel, nxt)

        return carry

    lax.fori_loop(0, _LEAF_CHUNKS // 2, outer, 0)

    c1b0 = wid * _C1_PER_W
    for t in range(3):
        base = c1b0 + t * _CH
        pltpu.sync_copy(c1_idx.at[pl.ds(base, _CH)], nidx_v.at[0])
        pltpu.async_copy(ntab.at[nidx_v.at[0]], nrow_v.at[0], sem_n).wait()
        pltpu.sync_copy(nrow_v.at[0], c1_out.at[pl.ds(base, _CH)])
        pltpu.sync_copy(e2_idx.at[pl.ds(base, _CH)], eidx_v.at[0])
        pltpu.async_copy(etab.at[eidx_v.at[0]], erow_v.at[0], sem_e).wait()
        pltpu.sync_copy(erow_v.at[0], e2_out.at[pl.ds(base, _CH)])

    c2b = wid * _C2_PER_W
    pltpu.sync_copy(c2_idx.at[pl.ds(c2b, _C2_PER_W)], cidx_v)
    pltpu.async_copy(ntab.at[cidx_v], crow_v, sem_c).wait()
    pltpu.sync_copy(crow_v, c2_out.at[pl.ds(c2b, _C2_PER_W)])


def _sc_gather(ntab, etab, leaf_idx, e1_idx, c1_idx, e2_idx, c2_idx):
    bf16 = jnp.bfloat16
    f32 = jnp.float32
    mesh = plsc.VectorSubcoreMesh(core_axis_name="c", subcore_axis_name="s")
    fn = functools.partial(
        pl.kernel,
        mesh=mesh,
        compiler_params=pltpu.CompilerParams(use_tc_tiling_on_sc=False),
        out_type=[
            jax.ShapeDtypeStruct((_N_LEAF, D), bf16),
            jax.ShapeDtypeStruct((_N_LEAF, DE), f32),
            jax.ShapeDtypeStruct((_C1_PAD, D), bf16),
            jax.ShapeDtypeStruct((_C1_PAD, DE), f32),
            jax.ShapeDtypeStruct((B, D), bf16),
        ],
        scratch_types=[
            pltpu.VMEM((2, _CH), jnp.int32),
            pltpu.VMEM((2, _CH, D), bf16),
            pltpu.VMEM((2, _CH), jnp.int32),
            pltpu.VMEM((2, _CH, DE), f32),
            pltpu.VMEM((_C2_PER_W,), jnp.int32),
            pltpu.VMEM((_C2_PER_W, D), bf16),
            pltpu.SemaphoreType.DMA,
            pltpu.SemaphoreType.DMA,
            pltpu.SemaphoreType.DMA,
        ],
    )(_sc_gather_body)
    return fn(ntab, etab, leaf_idx, e1_idx, c1_idx, e2_idx, c2_idx)


_COS_C = (0.9999994437071105, -19.739034397802136, 64.93061450604583,
          -85.29598723642509, 58.91264615607875, -21.283194092739)


def _fast_cos(x):
    f32 = jnp.float32
    r = x * f32(0.15915494309189535)
    r = r - jnp.round(r)
    u = r * r
    p = jnp.full_like(u, _COS_C[5])
    for c in _COS_C[4::-1]:
        p = p * u + f32(c)
    return p


def _attn_body(msz, cdt, leaf_ref, edge_ref, c_ref, nts_ref, nid_ref, tsc_ref,
               wq_ref, wkv_ref, w1_ref, b1_ref, w2_ref, b2_ref,
               tw_ref, tb_ref, o_ref):
    f32 = jnp.float32
    dot = functools.partial(jnp.dot, preferred_element_type=f32)
    tw = tw_ref[...]
    tb = tb_ref[...]
    cb = c_ref[...].astype(cdt)
    q = dot(cb, wq_ref[:D, :]) + dot(jnp.cos(tb).astype(cdt), wq_ref[D:, :])
    tscol = tsc_ref[...]
    dcol = jnp.stack([tscol - nts_ref[:, j:j + 1] for j in range(NB)], axis=0)
    dt_all = _fast_cos(dcol * tw[None] + tb[None])
    dt_all = dt_all.reshape(NB * msz, DT).astype(cdt)
    lf_all = leaf_ref[...].reshape(NB * msz, D).astype(cdt)
    eg_all = edge_ref[...].reshape(NB * msz, DE).astype(cdt)
    kv = (dot(lf_all, wkv_ref[:D, :]) + dot(eg_all, wkv_ref[D:D + DE, :])
          + dot(dt_all, wkv_ref[D + DE:, :]))
    inv_sqrt = f32(1.0 / np.sqrt(DH))
    sjs = []
    for j in range(NB):
        kj = lax.slice_in_dim(kv, j * msz, (j + 1) * msz, axis=0)[:, :DH]
        sjs.append(jnp.sum(q * kj, axis=1, keepdims=True))
    scores = jnp.concatenate(sjs, axis=1) * inv_sqrt
    scores = jnp.where(nid_ref[...] == 0, f32(-1e10), scores)
    mx = jnp.max(scores, axis=1, keepdims=True)
    e = jnp.exp(scores - mx)
    attn = e / jnp.sum(e, axis=1, keepdims=True)
    acc = jnp.zeros((msz, DH), f32)
    for j in range(NB):
        vj = lax.slice_in_dim(kv, j * msz, (j + 1) * msz, axis=0)[:, DH:]
        acc = acc + attn[:, j:j + 1] * vj
    agg = acc.astype(cdt)
    hmid = jnp.maximum(dot(agg, w1_ref[:DH, :]) + dot(cb, w1_ref[DH:, :])
                       + b1_ref[...], 0.0).astype(cdt)
    o_ref[...] = (dot(hmid, w2_ref[...]) + b2_ref[...]).astype(o_ref.dtype)


def _attn_level(leafT, edgeT, cx, nts, nid, tscol, wq, wkv, w1, b1, w2, b2,
                tw, tb, msz, n, cdt, odt):
    grid = (n // msz,)
    wspec = lambda shape: pl.BlockSpec(shape, lambda i: (0, 0))
    return pl.pallas_call(
        functools.partial(_attn_body, msz, cdt),
        grid=grid,
        in_specs=[
            pl.BlockSpec((NB, msz, D), lambda i: (0, i, 0)),
            pl.BlockSpec((NB, msz, DE), lambda i: (0, i, 0)),
            pl.BlockSpec((msz, D), lambda i: (i, 0)),
            pl.BlockSpec((msz, NB), lambda i: (i, 0)),
            pl.BlockSpec((msz, NB), lambda i: (i, 0)),
            pl.BlockSpec((msz, 1), lambda i: (i, 0)),
            wspec((D + DT, DH)),
            wspec((D + DE + DT, 2 * DH)),
            wspec((DH + D, DH)),
            wspec((1, DH)),
            wspec((DH, D)),
            wspec((1, D)),
            wspec((1, DT)),
            wspec((1, DT)),
        ],
        out_specs=pl.BlockSpec((msz, D), lambda i: (i, 0)),
        out_shape=jax.ShapeDtypeStruct((n, D), odt),
    )(leafT, edgeT, cx, nts, nid, tscol, wq, wkv, w1, b1, w2, b2, tw, tb)


def kernel(all_node_reprs, ts, neigh_ts_l2, neigh_ts_l1, node_emb, edge_emb,
           time_w, time_b, Wq1, Wk1, Wv1, W1_1, b1_1, W2_1, b2_1,
           Wq2, Wk2, Wv2, W1_2, b1_2, W2_2, b2_2,
           center_nids, neigh_nids_l2, neigh_eids_l2,
           neigh_nids_l1, neigh_eids_l1):
    f32 = jnp.float32
    i32 = jnp.int32
    bf16 = jnp.bfloat16

    nids1_3d = neigh_nids_l1.reshape(B, NB, NB)
    eids1_3d = neigh_eids_l1.reshape(B, NB, NB)
    nts1_3d = neigh_ts_l1.reshape(B, NB, NB)
    leaf_list = nids1_3d.transpose(2, 1, 0).reshape(-1)
    e1_list = eids1_3d.transpose(2, 1, 0).reshape(-1)
    c1_pad = jnp.zeros((_C1_PAD - _N_C1,), i32)
    c1_list = jnp.concatenate([neigh_nids_l2.T.reshape(-1), c1_pad])
    e2_list = jnp.concatenate([neigh_eids_l2.T.reshape(-1), c1_pad])
    nts1 = nts1_3d.transpose(1, 0, 2).reshape(B * NB, NB)
    nid1 = nids1_3d.transpose(1, 0, 2).reshape(B * NB, NB)
    ts1 = jnp.tile(ts, NB).reshape(B * NB, 1)

    tw2 = time_w.reshape(1, DT)
    tb2 = time_b.reshape(1, DT)
    b1_1r = b1_1.reshape(1, DH)
    b2_1r = b2_1.reshape(1, D)
    b1_2r = b1_2.reshape(1, DH)
    b2_2r = b2_2.reshape(1, D)

    comb = _combined_table(all_node_reprs, node_emb)

    leaf_rows, e1_rows, c1_rows, e2_rows, c2 = _sc_gather(
        comb, edge_emb, leaf_list, e1_list, c1_list, e2_list,
        center_nids.astype(i32))

    leafT = leaf_rows.reshape(NB, B * NB, D)
    e1T = e1_rows.reshape(NB, B * NB, DE)
    e2T = e2_rows[:_N_C1].reshape(NB, B, DE)

    wkv1 = jnp.concatenate([Wk1, Wv1], axis=1)
    h1 = _attn_level(leafT, e1T, c1_rows, nts1, nid1, ts1,
                     Wq1.astype(bf16), wkv1.astype(bf16),
                     W1_1.astype(bf16), b1_1r, W2_1.astype(bf16), b2_1r,
                     tw2, tb2, msz=256, n=B * NB, cdt=bf16, odt=bf16)

    wkv2 = jnp.concatenate([Wk2, Wv2], axis=1)
    neigh2T = h1.reshape(NB, B, D)
    out = _attn_level(neigh2T, e2T, c2, neigh_ts_l2, neigh_nids_l2,
                      ts.reshape(B, 1),
                      Wq2, wkv2, W1_2, b1_2r, W2_2, b2_2r, tw2, tb2,
                      msz=B, n=B, cdt=f32, odt=f32)
    return out
